# initial kernel scaffold (unmeasured)
import jax
import jax.numpy as jnp
from jax import lax
from jax.experimental import pallas as pl
from jax.experimental.pallas import tpu as pltpu

N_DEV = 16
B = 2
S = 256
D = 768
HQ = 4
DH = 64
C = HQ * DH
KV = 2 * C
SCALE = 0.125


def kernel(x, Wq, Wk, Wv, Wo):
    def body(x_ref, wq_ref, wk_ref, wv_ref, wo_ref, out_ref,
             kv_all, send_sems, recv_sems):
        my = lax.axis_index("i")
        left = lax.rem(my + N_DEV - 1, N_DEV)
        right = lax.rem(my + 1, N_DEV)

        barrier = pltpu.get_barrier_semaphore()
        for nbr in (left, right):
            pl.semaphore_signal(barrier, inc=1, device_id=(nbr,),
                                device_id_type=pl.DeviceIdType.MESH)
        pl.semaphore_wait(barrier, 2)

        row = lax.broadcasted_iota(jnp.float32, (S, C), 0)
        col = lax.broadcasted_iota(jnp.int32, (S, C), 1)
        j = (lax.rem(col, DH) // 2).astype(jnp.float32)
        inv = jnp.exp(j * (-2.0 * jnp.log(10000.0) / DH))
        pos = row + my.astype(jnp.float32) * float(S)
        ang = pos * inv
        cos_t = jnp.cos(ang)
        sin_t = jnp.sin(ang)

        r_i = lax.broadcasted_iota(jnp.int32, (C, C), 0)
        c_i = lax.broadcasted_iota(jnp.int32, (C, C), 1)
        c_even = lax.rem(c_i, 2) == 0
        R = jnp.where(c_even & (r_i == c_i + 1), -1.0,
                      jnp.where((~c_even) & (r_i == c_i - 1), 1.0, 0.0)
                      ).astype(jnp.float32)

        def rope(t):
            t_r = jnp.dot(t, R, preferred_element_type=jnp.float32)
            return t * cos_t + t_r * sin_t

        q_locals = []
        for b in range(B):
            xb = x_ref[b]
            qb = rope(jnp.dot(xb, wq_ref[:, :],
                              preferred_element_type=jnp.float32))
            kb = rope(jnp.dot(xb, wk_ref[:, :],
                              preferred_element_type=jnp.float32))
            vb = jnp.dot(xb, wv_ref[:, :],
                         preferred_element_type=jnp.float32)
            q_locals.append(qb)
            kv_all[b, pl.ds(0, S), pl.ds(0, C)] = kb
            kv_all[b, pl.ds(0, S), pl.ds(C, C)] = vb

        for h in range(N_DEV - 1):
            for b in range(B):
                rdma = pltpu.make_async_remote_copy(
                    src_ref=kv_all.at[b, pl.ds(h * S, S), :],
                    dst_ref=kv_all.at[b, pl.ds((h + 1) * S, S), :],
                    send_sem=send_sems.at[b, h % 2],
                    recv_sem=recv_sems.at[b, (h + 1) % 2],
                    device_id=(right,),
                    device_id_type=pl.DeviceIdType.MESH,
                )
                rdma.start()
                rdma.wait()

        for b in range(B):
            qb = q_locals[b]
            ctx_heads = []
            for hd in range(HQ):
                qbh = qb[:, hd * DH:(hd + 1) * DH]
                kall = kv_all[b, :, pl.ds(hd * DH, DH)]
                vall = kv_all[b, :, pl.ds(C + hd * DH, DH)]
                s = lax.dot_general(
                    qbh, kall, (((1,), (1,)), ((), ())),
                    preferred_element_type=jnp.float32) * SCALE
                m = jnp.max(s, axis=1, keepdims=True)
                e = jnp.exp(s - m)
                w = e / jnp.sum(e, axis=1, keepdims=True)
                ctx_heads.append(jnp.dot(w, vall,
                                         preferred_element_type=jnp.float32))
            ctx = jnp.concatenate(ctx_heads, axis=1)
            out_ref[b] = jnp.dot(ctx, wo_ref[:, :],
                                 preferred_element_type=jnp.float32)

    return pl.pallas_call(
        body,
        out_shape=jax.ShapeDtypeStruct((B, S, D), jnp.float32),
        in_specs=[pl.BlockSpec(memory_space=pltpu.VMEM)] * 5,
        out_specs=pl.BlockSpec(memory_space=pltpu.VMEM),
        scratch_shapes=[
            pltpu.VMEM((B, N_DEV * S, KV), jnp.float32),
            pltpu.SemaphoreType.DMA((B, 2)),
            pltpu.SemaphoreType.DMA((B, 2)),
        ],
        compiler_params=pltpu.CompilerParams(collective_id=0),
    )(x, Wq, Wk, Wv, Wo)


# baseline (device time: 254082 ns/iter reference)
import jax
import jax.numpy as jnp
from jax import lax
from jax.experimental import pallas as pl
from jax.experimental.pallas import tpu as pltpu

N_DEV = 16
B = 2
S = 256
D = 768
HQ = 4
DH = 64
C = HQ * DH
KV = 2 * C
SCALE = 0.125


def kernel(x, Wq, Wk, Wv, Wo):
    def body(x_ref, wq_ref, wk_ref, wv_ref, wo_ref, out_ref,
             kv_all, send_sems, recv_sems):
        my = lax.axis_index("i")
        left = lax.rem(my + N_DEV - 1, N_DEV)
        right = lax.rem(my + 1, N_DEV)

        barrier = pltpu.get_barrier_semaphore()
        for nbr in (left, right):
            pl.semaphore_signal(barrier, inc=1, device_id=(nbr,),
                                device_id_type=pl.DeviceIdType.MESH)
        pl.semaphore_wait(barrier, 2)

        row = lax.broadcasted_iota(jnp.int32, (S, C), 0).astype(jnp.float32)
        col = lax.broadcasted_iota(jnp.int32, (S, C), 1)
        j = (lax.rem(col, DH) // 2).astype(jnp.float32)
        inv = jnp.exp(j * (-2.0 * jnp.log(10000.0) / DH))
        pos = row + my.astype(jnp.float32) * float(S)
        ang = pos * inv
        cos_t = jnp.cos(ang)
        sin_t = jnp.sin(ang)

        r_i = lax.broadcasted_iota(jnp.int32, (C, C), 0)
        c_i = lax.broadcasted_iota(jnp.int32, (C, C), 1)
        c_even = lax.rem(c_i, 2) == 0
        R = jnp.where(c_even & (r_i == c_i + 1), -1.0,
                      jnp.where((~c_even) & (r_i == c_i - 1), 1.0, 0.0)
                      ).astype(jnp.float32)

        def rope(t):
            t_r = jnp.dot(t, R, preferred_element_type=jnp.float32)
            return t * cos_t + t_r * sin_t

        q_locals = []
        for b in range(B):
            xb = x_ref[b]
            qb = rope(jnp.dot(xb, wq_ref[:, :],
                              preferred_element_type=jnp.float32))
            kb = rope(jnp.dot(xb, wk_ref[:, :],
                              preferred_element_type=jnp.float32))
            vb = jnp.dot(xb, wv_ref[:, :],
                         preferred_element_type=jnp.float32)
            q_locals.append(qb)
            kv_all[b, pl.ds(0, S), pl.ds(0, C)] = kb
            kv_all[b, pl.ds(0, S), pl.ds(C, C)] = vb

        for h in range(N_DEV - 1):
            for b in range(B):
                rdma = pltpu.make_async_remote_copy(
                    src_ref=kv_all.at[b, pl.ds(h * S, S), :],
                    dst_ref=kv_all.at[b, pl.ds((h + 1) * S, S), :],
                    send_sem=send_sems.at[b, h % 2],
                    recv_sem=recv_sems.at[b, (h + 1) % 2],
                    device_id=(right,),
                    device_id_type=pl.DeviceIdType.MESH,
                )
                rdma.start()
                rdma.wait()

        for b in range(B):
            qb = q_locals[b]
            ctx_heads = []
            for hd in range(HQ):
                qbh = qb[:, hd * DH:(hd + 1) * DH]
                kall = kv_all[b, :, pl.ds(hd * DH, DH)]
                vall = kv_all[b, :, pl.ds(C + hd * DH, DH)]
                s = lax.dot_general(
                    qbh, kall, (((1,), (1,)), ((), ())),
                    preferred_element_type=jnp.float32) * SCALE
                m = jnp.max(s, axis=1, keepdims=True)
                e = jnp.exp(s - m)
                w = e / jnp.sum(e, axis=1, keepdims=True)
                ctx_heads.append(jnp.dot(w, vall,
                                         preferred_element_type=jnp.float32))
            ctx = jnp.concatenate(ctx_heads, axis=1)
            out_ref[b] = jnp.dot(ctx, wo_ref[:, :],
                                 preferred_element_type=jnp.float32)

    return pl.pallas_call(
        body,
        out_shape=jax.ShapeDtypeStruct((B, S, D), jnp.float32),
        in_specs=[pl.BlockSpec(memory_space=pltpu.VMEM)] * 5,
        out_specs=pl.BlockSpec(memory_space=pltpu.VMEM),
        scratch_shapes=[
            pltpu.VMEM((B, N_DEV * S, KV), jnp.float32),
            pltpu.SemaphoreType.DMA((B, 2)),
            pltpu.SemaphoreType.DMA((B, 2)),
        ],
        compiler_params=pltpu.CompilerParams(collective_id=0),
    )(x, Wq, Wk, Wv, Wo)


# device time: 147714 ns/iter; 1.7201x vs baseline; 1.7201x over previous
import jax
import jax.numpy as jnp
from jax import lax
from jax.experimental import pallas as pl
from jax.experimental.pallas import tpu as pltpu

N_DEV = 16
NR = 8
NL = 7
B = 2
S = 256
D = 768
HQ = 4
DH = 64
C = HQ * DH
KV = 2 * C
SCALE = 0.125


def kernel(x, Wq, Wk, Wv, Wo):
    def body(x_ref, wq_ref, wk_ref, wv_ref, wo_ref, out_ref,
             kv_all, ssem_r, rsem_r, ssem_l, rsem_l, credit_r, credit_l):
        my = lax.axis_index("i")
        left = lax.rem(my + N_DEV - 1, N_DEV)
        right = lax.rem(my + 1, N_DEV)

        barrier = pltpu.get_barrier_semaphore()
        for nbr in (left, right):
            pl.semaphore_signal(barrier, inc=1, device_id=(nbr,),
                                device_id_type=pl.DeviceIdType.MESH)
        pl.semaphore_wait(barrier, 2)

        row = lax.broadcasted_iota(jnp.int32, (S, C), 0).astype(jnp.float32)
        col = lax.broadcasted_iota(jnp.int32, (S, C), 1)
        j = (lax.rem(col, DH) // 2).astype(jnp.float32)
        inv = jnp.exp(j * (-2.0 * jnp.log(10000.0) / DH))
        pos = row + my.astype(jnp.float32) * float(S)
        ang = pos * inv
        cos_t = jnp.cos(ang)
        sin_t = jnp.sin(ang)

        r_i = lax.broadcasted_iota(jnp.int32, (C, C), 0)
        c_i = lax.broadcasted_iota(jnp.int32, (C, C), 1)
        c_even = lax.rem(c_i, 2) == 0
        R = jnp.where(c_even & (r_i == c_i + 1), -1.0,
                      jnp.where((~c_even) & (r_i == c_i - 1), 1.0, 0.0)
                      ).astype(jnp.float32)

        def rope(t):
            t_r = jnp.dot(t, R, preferred_element_type=jnp.float32)
            return t * cos_t + t_r * sin_t

        q_locals = []
        for b in range(B):
            xb = x_ref[b]
            qb = rope(jnp.dot(xb, wq_ref[:, :],
                              preferred_element_type=jnp.float32))
            kb = rope(jnp.dot(xb, wk_ref[:, :],
                              preferred_element_type=jnp.float32))
            vb = jnp.dot(xb, wv_ref[:, :],
                         preferred_element_type=jnp.float32)
            q_locals.append(qb)
            kv_all[0, b, :, pl.ds(0, C)] = kb
            kv_all[0, b, :, pl.ds(C, C)] = vb

        def rcopy(src_slot, dst_slot, ssem, rsem, target):
            return pltpu.make_async_remote_copy(
                src_ref=kv_all.at[src_slot],
                dst_ref=kv_all.at[dst_slot],
                send_sem=ssem, recv_sem=rsem,
                device_id=(target,),
                device_id_type=pl.DeviceIdType.MESH,
            )

        for h in range(NR):
            r = rcopy(h, h + 1,
                      ssem_r.at[h % 2], rsem_r.at[(h + 1) % 2], right)
            r.start()
            if h < NL:
                l = rcopy(8 + h if h else 0, 9 + h,
                          ssem_l.at[h % 2], rsem_l.at[(h + 1) % 2], left)
                l.start()
                r.wait()
                l.wait()
            else:
                r.wait()

        for b in range(B):
            qb = q_locals[b]
            ctx_heads = []
            for hd in range(HQ):
                qbh = qb[:, hd * DH:(hd + 1) * DH]
                kall = jnp.concatenate(
                    [kv_all[t, b, :, pl.ds(hd * DH, DH)]
                     for t in range(N_DEV)], axis=0)
                vall = jnp.concatenate(
                    [kv_all[t, b, :, pl.ds(C + hd * DH, DH)]
                     for t in range(N_DEV)], axis=0)
                s = lax.dot_general(
                    qbh, kall, (((1,), (1,)), ((), ())),
                    preferred_element_type=jnp.float32) * SCALE
                m = jnp.max(s, axis=1, keepdims=True)
                e = jnp.exp(s - m)
                w = e / jnp.sum(e, axis=1, keepdims=True)
                ctx_heads.append(jnp.dot(w, vall,
                                         preferred_element_type=jnp.float32))
            ctx = jnp.concatenate(ctx_heads, axis=1)
            out_ref[b] = jnp.dot(ctx, wo_ref[:, :],
                                 preferred_element_type=jnp.float32)

    return pl.pallas_call(
        body,
        out_shape=jax.ShapeDtypeStruct((B, S, D), jnp.float32),
        in_specs=[pl.BlockSpec(memory_space=pltpu.VMEM)] * 5,
        out_specs=pl.BlockSpec(memory_space=pltpu.VMEM),
        scratch_shapes=[
            pltpu.VMEM((N_DEV, B, S, KV), jnp.float32),
            pltpu.SemaphoreType.DMA((2,)),
            pltpu.SemaphoreType.DMA((2,)),
            pltpu.SemaphoreType.DMA((2,)),
            pltpu.SemaphoreType.DMA((2,)),
            pltpu.SemaphoreType.REGULAR,
            pltpu.SemaphoreType.REGULAR,
        ],
        compiler_params=pltpu.CompilerParams(collective_id=0),
    )(x, Wq, Wk, Wv, Wo)


# device time: 132102 ns/iter; 1.9234x vs baseline; 1.1182x over previous
import jax
import jax.numpy as jnp
from jax import lax
from jax.experimental import pallas as pl
from jax.experimental.pallas import tpu as pltpu

N_DEV = 16
NR = 8
NL = 7
B = 2
S = 256
D = 768
HQ = 4
DH = 64
C = HQ * DH
KV = 2 * C
SCALE = 0.125


def kernel(x, Wq, Wk, Wv, Wo):
    def body(x_ref, wq_ref, wk_ref, wv_ref, wo_ref, out_ref,
             kv_all, ssem_r, rsem_r, ssem_l, rsem_l):
        my = lax.axis_index("i")
        left = lax.rem(my + N_DEV - 1, N_DEV)
        right = lax.rem(my + 1, N_DEV)

        barrier = pltpu.get_barrier_semaphore()
        for nbr in (left, right):
            pl.semaphore_signal(barrier, inc=1, device_id=(nbr,),
                                device_id_type=pl.DeviceIdType.MESH)
        pl.semaphore_wait(barrier, 2)

        row = lax.broadcasted_iota(jnp.int32, (S, C), 0).astype(jnp.float32)
        col = lax.broadcasted_iota(jnp.int32, (S, C), 1)
        j = (lax.rem(col, DH) // 2).astype(jnp.float32)
        inv = jnp.exp(j * (-2.0 * jnp.log(10000.0) / DH))
        pos = row + my.astype(jnp.float32) * float(S)
        ang = pos * inv
        cos_t = jnp.cos(ang)
        sin_t = jnp.sin(ang)

        r_i = lax.broadcasted_iota(jnp.int32, (C, C), 0)
        c_i = lax.broadcasted_iota(jnp.int32, (C, C), 1)
        c_even = lax.rem(c_i, 2) == 0
        R = jnp.where(c_even & (r_i == c_i + 1), -1.0,
                      jnp.where((~c_even) & (r_i == c_i - 1), 1.0, 0.0)
                      ).astype(jnp.float32)

        def rope(t):
            t_r = jnp.dot(t, R, preferred_element_type=jnp.float32)
            return t * cos_t + t_r * sin_t

        for b in range(B):
            xb = x_ref[b]
            kb = rope(jnp.dot(xb, wk_ref[:, :],
                              preferred_element_type=jnp.float32))
            vb = jnp.dot(xb, wv_ref[:, :],
                         preferred_element_type=jnp.float32)
            kv_all[0, b, :, pl.ds(0, C)] = kb
            kv_all[0, b, :, pl.ds(C, C)] = vb

        def rcopy(src_slot, dst_slot, ssem, rsem, target):
            return pltpu.make_async_remote_copy(
                src_ref=kv_all.at[src_slot],
                dst_ref=kv_all.at[dst_slot],
                send_sem=ssem, recv_sem=rsem,
                device_id=(target,),
                device_id_type=pl.DeviceIdType.MESH,
            )

        q_heads = [[None] * HQ for _ in range(B)]
        m_st = [[None] * HQ for _ in range(B)]
        l_st = [[None] * HQ for _ in range(B)]
        acc_st = [[None] * HQ for _ in range(B)]

        def consume(slot, first=False):
            for b in range(B):
                for hd in range(HQ):
                    kc = kv_all[slot, b, :, pl.ds(hd * DH, DH)]
                    vc = kv_all[slot, b, :, pl.ds(C + hd * DH, DH)]
                    s = lax.dot_general(
                        q_heads[b][hd], kc, (((1,), (1,)), ((), ())),
                        preferred_element_type=jnp.float32) * SCALE
                    mc = jnp.max(s, axis=1, keepdims=True)
                    if first:
                        m_new = mc
                        p = jnp.exp(s - m_new)
                        l_new = jnp.sum(p, axis=1, keepdims=True)
                        a_new = jnp.dot(p, vc,
                                        preferred_element_type=jnp.float32)
                    else:
                        m_new = jnp.maximum(m_st[b][hd], mc)
                        alpha = jnp.exp(m_st[b][hd] - m_new)
                        p = jnp.exp(s - m_new)
                        l_new = l_st[b][hd] * alpha + jnp.sum(
                            p, axis=1, keepdims=True)
                        a_new = acc_st[b][hd] * alpha + jnp.dot(
                            p, vc, preferred_element_type=jnp.float32)
                    m_st[b][hd] = m_new
                    l_st[b][hd] = l_new
                    acc_st[b][hd] = a_new

        for h in range(NR):
            r = rcopy(h, h + 1, ssem_r.at[h % 2], rsem_r.at[h], right)
            r.start()
            l = None
            if h < NL:
                l = rcopy(8 + h if h else 0, 9 + h,
                          ssem_l.at[h % 2], rsem_l.at[h], left)
                l.start()

            if h == 0:
                for b in range(B):
                    qb = rope(jnp.dot(x_ref[b], wq_ref[:, :],
                                      preferred_element_type=jnp.float32))
                    for hd in range(HQ):
                        q_heads[b][hd] = qb[:, hd * DH:(hd + 1) * DH]
                consume(0, first=True)
            else:
                consume(h)
                consume(8 + h)

            r.wait()
            if l is not None:
                l.wait()

        consume(NR)

        for b in range(B):
            ctx = jnp.concatenate(
                [acc_st[b][hd] / l_st[b][hd] for hd in range(HQ)], axis=1)
            out_ref[b] = jnp.dot(ctx, wo_ref[:, :],
                                 preferred_element_type=jnp.float32)

    return pl.pallas_call(
        body,
        out_shape=jax.ShapeDtypeStruct((B, S, D), jnp.float32),
        in_specs=[pl.BlockSpec(memory_space=pltpu.VMEM)] * 5,
        out_specs=pl.BlockSpec(memory_space=pltpu.VMEM),
        scratch_shapes=[
            pltpu.VMEM((N_DEV, B, S, KV), jnp.float32),
            pltpu.SemaphoreType.DMA((2,)),
            pltpu.SemaphoreType.DMA((NR,)),
            pltpu.SemaphoreType.DMA((2,)),
            pltpu.SemaphoreType.DMA((NL,)),
        ],
        compiler_params=pltpu.CompilerParams(collective_id=0),
    )(x, Wq, Wk, Wv, Wo)


# device time: 83926 ns/iter; 3.0275x vs baseline; 1.5740x over previous
import jax
import jax.numpy as jnp
from jax import lax
from jax.experimental import pallas as pl
from jax.experimental.pallas import tpu as pltpu

N_DEV = 16
NR = 8
NL = 7
B = 2
S = 256
D = 768
HQ = 4
DH = 64
C = HQ * DH
KV = 2 * C
SCALE = 0.125


def kernel(x, Wq, Wk, Wv, Wo):
    def body(x_ref, wq_ref, wk_ref, wv_ref, wo_ref, out_ref,
             kv_all, ssem_r, rsem_r, ssem_l, rsem_l):
        my = lax.axis_index("i")
        left = lax.rem(my + N_DEV - 1, N_DEV)
        right = lax.rem(my + 1, N_DEV)

        barrier = pltpu.get_barrier_semaphore()
        for nbr in (left, right):
            pl.semaphore_signal(barrier, inc=1, device_id=(nbr,),
                                device_id_type=pl.DeviceIdType.MESH)
        pl.semaphore_wait(barrier, 2)

        row = lax.broadcasted_iota(jnp.int32, (S, C), 0).astype(jnp.float32)
        col = lax.broadcasted_iota(jnp.int32, (S, C), 1)
        j = (lax.rem(col, DH) // 2).astype(jnp.float32)
        inv = jnp.exp(j * (-2.0 * jnp.log(10000.0) / DH))
        pos = row + my.astype(jnp.float32) * float(S)
        ang = pos * inv
        cos_t = jnp.cos(ang)
        sin_t = jnp.sin(ang)

        r_i = lax.broadcasted_iota(jnp.int32, (C, C), 0)
        c_i = lax.broadcasted_iota(jnp.int32, (C, C), 1)
        c_even = lax.rem(c_i, 2) == 0
        R = jnp.where(c_even & (r_i == c_i + 1), -1.0,
                      jnp.where((~c_even) & (r_i == c_i - 1), 1.0, 0.0)
                      ).astype(jnp.float32)

        def rope(t):
            t_r = jnp.dot(t, R, preferred_element_type=jnp.float32)
            return t * cos_t + t_r * sin_t

        for b in range(B):
            xb = x_ref[b]
            kb = rope(jnp.dot(xb, wk_ref[:, :],
                              preferred_element_type=jnp.float32))
            vb = jnp.dot(xb, wv_ref[:, :],
                         preferred_element_type=jnp.float32)
            kv_all[0, b, :, pl.ds(0, C)] = kb.astype(jnp.bfloat16)
            kv_all[0, b, :, pl.ds(C, C)] = vb.astype(jnp.bfloat16)

        def rcopy(src_slot, dst_slot, ssem, rsem, target):
            return pltpu.make_async_remote_copy(
                src_ref=kv_all.at[src_slot],
                dst_ref=kv_all.at[dst_slot],
                send_sem=ssem, recv_sem=rsem,
                device_id=(target,),
                device_id_type=pl.DeviceIdType.MESH,
            )

        q_heads = [[None] * HQ for _ in range(B)]
        m_st = [[None] * HQ for _ in range(B)]
        l_st = [[None] * HQ for _ in range(B)]
        acc_st = [[None] * HQ for _ in range(B)]

        def consume(slot, first=False):
            for b in range(B):
                for hd in range(HQ):
                    kc = kv_all[slot, b, :, pl.ds(hd * DH, DH)].astype(
                        jnp.float32)
                    vc = kv_all[slot, b, :, pl.ds(C + hd * DH, DH)].astype(
                        jnp.float32)
                    s = lax.dot_general(
                        q_heads[b][hd], kc, (((1,), (1,)), ((), ())),
                        preferred_element_type=jnp.float32) * SCALE
                    mc = jnp.max(s, axis=1, keepdims=True)
                    if first:
                        m_new = mc
                        p = jnp.exp(s - m_new)
                        l_new = jnp.sum(p, axis=1, keepdims=True)
                        a_new = jnp.dot(p, vc,
                                        preferred_element_type=jnp.float32)
                    else:
                        m_new = jnp.maximum(m_st[b][hd], mc)
                        alpha = jnp.exp(m_st[b][hd] - m_new)
                        p = jnp.exp(s - m_new)
                        l_new = l_st[b][hd] * alpha + jnp.sum(
                            p, axis=1, keepdims=True)
                        a_new = acc_st[b][hd] * alpha + jnp.dot(
                            p, vc, preferred_element_type=jnp.float32)
                    m_st[b][hd] = m_new
                    l_st[b][hd] = l_new
                    acc_st[b][hd] = a_new

        for h in range(NR):
            r = rcopy(h, h + 1, ssem_r.at[h % 2], rsem_r.at[h], right)
            r.start()
            l = None
            if h < NL:
                l = rcopy(8 + h if h else 0, 9 + h,
                          ssem_l.at[h % 2], rsem_l.at[h], left)
                l.start()

            if h == 0:
                for b in range(B):
                    qb = rope(jnp.dot(x_ref[b], wq_ref[:, :],
                                      preferred_element_type=jnp.float32))
                    for hd in range(HQ):
                        q_heads[b][hd] = qb[:, hd * DH:(hd + 1) * DH]
                consume(0, first=True)
            else:
                consume(h)
                consume(8 + h)

            r.wait()
            if l is not None:
                l.wait()

        consume(NR)

        for b in range(B):
            ctx = jnp.concatenate(
                [acc_st[b][hd] / l_st[b][hd] for hd in range(HQ)], axis=1)
            out_ref[b] = jnp.dot(ctx, wo_ref[:, :],
                                 preferred_element_type=jnp.float32)

    return pl.pallas_call(
        body,
        out_shape=jax.ShapeDtypeStruct((B, S, D), jnp.float32),
        in_specs=[pl.BlockSpec(memory_space=pltpu.VMEM)] * 5,
        out_specs=pl.BlockSpec(memory_space=pltpu.VMEM),
        scratch_shapes=[
            pltpu.VMEM((N_DEV, B, S, KV), jnp.bfloat16),
            pltpu.SemaphoreType.DMA((2,)),
            pltpu.SemaphoreType.DMA((NR,)),
            pltpu.SemaphoreType.DMA((2,)),
            pltpu.SemaphoreType.DMA((NL,)),
        ],
        compiler_params=pltpu.CompilerParams(collective_id=0),
    )(x, Wq, Wk, Wv, Wo)
